# baseline (speedup 1.0000x reference)
import functools

import jax
import jax.numpy as jnp
from jax import lax
from jax.experimental import pallas as pl
from jax.experimental.pallas import tpu as pltpu

Q = 1024
D = 32
K = 5
C = 10
BN = 2048
S = 16
NL = BN // S
INF = float("inf")


def _dist2(x_ref, f_ref, g, n_total):
    xx = x_ref[:]
    xn = jnp.sum(xx * xx, axis=1, keepdims=True)
    f = f_ref[:]
    yn = jnp.sum(f * f, axis=1).reshape(1, BN)
    col = lax.broadcasted_iota(jnp.int32, (1, BN), 1)
    gcol = g * BN + col
    yn = jnp.where(gcol < n_total, yn, INF)
    prod = lax.dot_general(xx, f, (((1,), (1,)), ((), ())),
                           preferred_element_type=jnp.float32)
    d2 = jnp.maximum(xn + yn - 2.0 * prod, 0.0)
    return d2, gcol


def _body_a(n_total, n_blocks, x_ref, f_ref, out_ref, runv):
    g = pl.program_id(0)

    @pl.when(g == 0)
    def _init():
        runv[:] = jnp.full((Q, 16), INF, jnp.float32)

    d2, _ = _dist2(x_ref, f_ref, g, n_total)
    cm = jnp.min(d2.reshape(Q, S, NL), axis=1)
    c = jnp.concatenate([runv[:, :8], cm], axis=1)
    ms = []
    for j in range(K):
        m = jnp.min(c, axis=1, keepdims=True)
        ms.append(m)
        if j < K - 1:
            c = jnp.where(c <= m, INF, c)
    inf1 = jnp.full((Q, 1), INF, jnp.float32)
    runv[:] = jnp.concatenate(ms + [inf1] * 11, axis=1)

    @pl.when(g == n_blocks - 1)
    def _finish():
        out_ref[:] = ms[K - 1]


def _extract5(dv, pv, width_pad):
    out_d, out_p = [], []
    for j in range(K):
        m = jnp.min(dv, axis=1, keepdims=True)
        sel = jnp.min(jnp.where(dv == m, pv, INF), axis=1, keepdims=True)
        out_d.append(m)
        out_p.append(sel)
        if j < K - 1:
            dv = jnp.where(pv == sel, INF, dv)
    inf1 = jnp.full((Q, 1), INF, jnp.float32)
    return (jnp.concatenate(out_d + [inf1] * width_pad, axis=1),
            jnp.concatenate(out_p + [inf1] * width_pad, axis=1))


def _body_b(n_total, n_blocks, x_ref, f_ref, lab_ref, t_ref, out_ref,
            run_d, run_p):
    g = pl.program_id(0)

    @pl.when(g == 0)
    def _init():
        run_d[:] = jnp.full((Q, 16), INF, jnp.float32)
        run_p[:] = jnp.full((Q, 16), INF, jnp.float32)

    d2, gcol = _dist2(x_ref, f_ref, g, n_total)
    lab = lab_ref[0]
    pk = (gcol * 16 + lab).astype(jnp.float32)

    t = (t_ref[:] * 1.00002).reshape(Q, 1, 1)
    d3 = d2.reshape(Q, S, NL)
    pk3 = jnp.broadcast_to(pk.reshape(1, S, NL), (Q, S, NL))
    hit = d3 <= t
    nhit = jnp.max(jnp.sum(hit.astype(jnp.float32), axis=1))

    def cheap(_):
        hm = jnp.where(hit, d3, INF)
        cm = jnp.min(hm, axis=1)
        cp = jnp.min(jnp.where(hm == cm[:, None, :], pk3, INF), axis=1)
        return _extract5(cm, cp, 3)

    def full(_):
        return _extract5(d2, jnp.broadcast_to(pk, (Q, BN)), 3)

    bw_d, bw_p = lax.cond(nhit > 1.5, full, cheap, 0)

    cd = jnp.concatenate([run_d[:, :8], bw_d[:, :8]], axis=1)
    cp2 = jnp.concatenate([run_p[:, :8], bw_p[:, :8]], axis=1)
    n_d, n_p = [], []
    for j in range(K):
        m = jnp.min(cd, axis=1, keepdims=True)
        sel = jnp.min(jnp.where(cd == m, cp2, INF), axis=1, keepdims=True)
        n_d.append(m)
        n_p.append(sel)
        if j < K - 1:
            cd = jnp.where(cp2 == sel, INF, cd)
    inf1 = jnp.full((Q, 1), INF, jnp.float32)
    run_d[:] = jnp.concatenate(n_d + [inf1] * 11, axis=1)
    run_p[:] = jnp.concatenate(n_p + [inf1] * 11, axis=1)

    @pl.when(g == n_blocks - 1)
    def _finish():
        dist = [jnp.sqrt(v) for v in n_d]
        s0 = -dist[0]
        e = [jnp.exp(-v - s0) for v in dist]
        tot = e[0] + e[1] + e[2] + e[3] + e[4]
        iota_c = lax.broadcasted_iota(jnp.int32, (Q, C), 1)
        o = jnp.zeros((Q, C), jnp.float32)
        for j in range(K):
            labj = n_p[j].astype(jnp.int32) & 15
            o = o + (e[j] / tot) * (labj == iota_c).astype(jnp.float32)
        out_ref[:] = o


def kernel(x, train_features, train_labels):
    n = train_features.shape[0]
    g = -(-n // BN)
    npad = g * BN
    f = jnp.pad(train_features, ((0, npad - n), (0, 0)))
    labs = jnp.pad(train_labels, (0, npad - n)).reshape(g, 1, BN)

    t = pl.pallas_call(
        functools.partial(_body_a, n, g),
        grid=(g,),
        in_specs=[
            pl.BlockSpec((Q, D), lambda i: (0, 0)),
            pl.BlockSpec((BN, D), lambda i: (i, 0)),
        ],
        out_specs=pl.BlockSpec((Q, 1), lambda i: (0, 0)),
        out_shape=jax.ShapeDtypeStruct((Q, 1), jnp.float32),
        scratch_shapes=[pltpu.VMEM((Q, 16), jnp.float32)],
        compiler_params=pltpu.CompilerParams(
            dimension_semantics=("arbitrary",),
        ),
    )(x, f)

    return pl.pallas_call(
        functools.partial(_body_b, n, g),
        grid=(g,),
        in_specs=[
            pl.BlockSpec((Q, D), lambda i: (0, 0)),
            pl.BlockSpec((BN, D), lambda i: (i, 0)),
            pl.BlockSpec((1, 1, BN), lambda i: (i, 0, 0)),
            pl.BlockSpec((Q, 1), lambda i: (0, 0)),
        ],
        out_specs=pl.BlockSpec((Q, C), lambda i: (0, 0)),
        out_shape=jax.ShapeDtypeStruct((Q, C), jnp.float32),
        scratch_shapes=[
            pltpu.VMEM((Q, 16), jnp.float32),
            pltpu.VMEM((Q, 16), jnp.float32),
        ],
        compiler_params=pltpu.CompilerParams(
            dimension_semantics=("arbitrary",),
        ),
    )(x, f, labs, t)


# speedup vs baseline: 1.0665x; 1.0665x over previous
import functools

import jax
import jax.numpy as jnp
from jax import lax
from jax.experimental import pallas as pl
from jax.experimental.pallas import tpu as pltpu

Q = 1024
D = 32
K = 5
C = 10
BN = 2048
S = 16
NL = BN // S
INF = float("inf")


def _dist2(x_ref, f_ref, g, n_total):
    xx = x_ref[:]
    xn = jnp.sum(xx * xx, axis=1, keepdims=True)
    f = f_ref[:]
    yn = jnp.sum(f * f, axis=1).reshape(1, BN)
    col = lax.broadcasted_iota(jnp.int32, (1, BN), 1)
    gcol = g * BN + col
    yn = jnp.where(gcol < n_total, yn, INF)
    prod = lax.dot_general(xx, f, (((1,), (1,)), ((), ())),
                           preferred_element_type=jnp.float32)
    d2 = jnp.maximum(xn + yn - 2.0 * prod, 0.0)
    return d2, gcol


def _finish_out(n_d, n_p, out_ref):
    dist = [jnp.sqrt(v) for v in n_d]
    s0 = -dist[0]
    e = [jnp.exp(-v - s0) for v in dist]
    tot = e[0] + e[1] + e[2] + e[3] + e[4]
    iota_c = lax.broadcasted_iota(jnp.int32, (Q, C), 1)
    o = jnp.zeros((Q, C), jnp.float32)
    for j in range(K):
        labj = n_p[j].astype(jnp.int32) & 15
        o = o + (e[j] / tot) * (labj == iota_c).astype(jnp.float32)
    out_ref[:] = o


def _merge_run(bw_d, bw_p, run_d, run_p):
    cd = jnp.concatenate([run_d[:, :8], bw_d[:, :8]], axis=1)
    cp = jnp.concatenate([run_p[:, :8], bw_p[:, :8]], axis=1)
    n_d, n_p = [], []
    for j in range(K):
        m = jnp.min(cd, axis=1, keepdims=True)
        sel = jnp.min(jnp.where(cd == m, cp, INF), axis=1, keepdims=True)
        n_d.append(m)
        n_p.append(sel)
        if j < K - 1:
            cd = jnp.where(cp == sel, INF, cd)
    inf1 = jnp.full((Q, 1), INF, jnp.float32)
    run_d[:] = jnp.concatenate(n_d + [inf1] * 11, axis=1)
    run_p[:] = jnp.concatenate(n_p + [inf1] * 11, axis=1)
    return n_d, n_p


def _extract5(dv, pv, width_pad):
    out_d, out_p = [], []
    for j in range(K):
        m = jnp.min(dv, axis=1, keepdims=True)
        sel = jnp.min(jnp.where(dv == m, pv, INF), axis=1, keepdims=True)
        out_d.append(m)
        out_p.append(sel)
        if j < K - 1:
            dv = jnp.where(pv == sel, INF, dv)
    inf1 = jnp.full((Q, 1), INF, jnp.float32)
    return (jnp.concatenate(out_d + [inf1] * width_pad, axis=1),
            jnp.concatenate(out_p + [inf1] * width_pad, axis=1))


def _body_a(n_total, n_blocks, x_ref, f_ref, out_ref, runv):
    g = pl.program_id(0)

    @pl.when(g == 0)
    def _init():
        runv[:] = jnp.full((Q, 16), INF, jnp.float32)

    d2, _ = _dist2(x_ref, f_ref, g, n_total)
    cm = jnp.min(d2.reshape(Q, S, NL), axis=1)
    c = jnp.concatenate([runv[:, :8], cm], axis=1)
    ms = []
    for j in range(K):
        m = jnp.min(c, axis=1, keepdims=True)
        ms.append(m)
        if j < K - 1:
            c = jnp.where(c <= m, INF, c)
    inf1 = jnp.full((Q, 1), INF, jnp.float32)
    runv[:] = jnp.concatenate(ms + [inf1] * 11, axis=1)

    @pl.when(g == n_blocks - 1)
    def _finish():
        out_ref[:] = ms[K - 1]


def _body_b(n_total, n_blocks, x_ref, f_ref, lab_ref, t_ref, out_ref,
            flag_ref, run_d, run_p):
    g = pl.program_id(0)

    @pl.when(g == 0)
    def _init():
        run_d[:] = jnp.full((Q, 16), INF, jnp.float32)
        run_p[:] = jnp.full((Q, 16), INF, jnp.float32)
        flag_ref[:] = jnp.zeros((1, NL), jnp.float32)

    d2, gcol = _dist2(x_ref, f_ref, g, n_total)
    lab = lab_ref[0]
    pk = (gcol * 16 + lab).astype(jnp.float32)

    t = (t_ref[:] * 1.00002).reshape(Q, 1, 1)
    d3 = d2.reshape(Q, S, NL)
    pk3 = jnp.broadcast_to(pk.reshape(1, S, NL), (Q, S, NL))
    hm = jnp.where(d3 <= t, d3, INF)
    cnt = jnp.sum((hm < INF).astype(jnp.float32), axis=1)
    flag_ref[:] = jnp.maximum(flag_ref[:],
                              jnp.max(cnt, axis=0, keepdims=True))

    cm = jnp.min(hm, axis=1)
    cp = jnp.min(jnp.where(hm == cm[:, None, :], pk3, INF), axis=1)
    hm2 = jnp.where(pk3 == cp[:, None, :], INF, hm)
    cm2 = jnp.min(hm2, axis=1)
    cp2 = jnp.min(jnp.where(hm2 == cm2[:, None, :], pk3, INF), axis=1)

    bw_d, bw_p = _extract5(jnp.concatenate([cm, cm2], axis=1),
                           jnp.concatenate([cp, cp2], axis=1), 3)
    n_d, n_p = _merge_run(bw_d, bw_p, run_d, run_p)

    @pl.when(g == n_blocks - 1)
    def _finish():
        _finish_out(n_d, n_p, out_ref)


def _body_full(n_total, n_blocks, x_ref, f_ref, lab_ref, out_ref,
               run_d, run_p):
    g = pl.program_id(0)

    @pl.when(g == 0)
    def _init():
        run_d[:] = jnp.full((Q, 16), INF, jnp.float32)
        run_p[:] = jnp.full((Q, 16), INF, jnp.float32)

    d2, gcol = _dist2(x_ref, f_ref, g, n_total)
    lab = lab_ref[0]
    pk = (gcol * 16 + lab).astype(jnp.float32)
    bw_d, bw_p = _extract5(d2, jnp.broadcast_to(pk, (Q, BN)), 3)
    n_d, n_p = _merge_run(bw_d, bw_p, run_d, run_p)

    @pl.when(g == n_blocks - 1)
    def _finish():
        _finish_out(n_d, n_p, out_ref)


def kernel(x, train_features, train_labels):
    n = train_features.shape[0]
    g = -(-n // BN)
    npad = g * BN
    f = jnp.pad(train_features, ((0, npad - n), (0, 0)))
    labs = jnp.pad(train_labels, (0, npad - n)).reshape(g, 1, BN)
    cparams = pltpu.CompilerParams(dimension_semantics=("arbitrary",))

    t = pl.pallas_call(
        functools.partial(_body_a, n, g),
        grid=(g,),
        in_specs=[
            pl.BlockSpec((Q, D), lambda i: (0, 0)),
            pl.BlockSpec((BN, D), lambda i: (i, 0)),
        ],
        out_specs=pl.BlockSpec((Q, 1), lambda i: (0, 0)),
        out_shape=jax.ShapeDtypeStruct((Q, 1), jnp.float32),
        scratch_shapes=[pltpu.VMEM((Q, 16), jnp.float32)],
        compiler_params=cparams,
    )(x, f)

    out, flag = pl.pallas_call(
        functools.partial(_body_b, n, g),
        grid=(g,),
        in_specs=[
            pl.BlockSpec((Q, D), lambda i: (0, 0)),
            pl.BlockSpec((BN, D), lambda i: (i, 0)),
            pl.BlockSpec((1, 1, BN), lambda i: (i, 0, 0)),
            pl.BlockSpec((Q, 1), lambda i: (0, 0)),
        ],
        out_specs=[
            pl.BlockSpec((Q, C), lambda i: (0, 0)),
            pl.BlockSpec((1, NL), lambda i: (0, 0)),
        ],
        out_shape=[
            jax.ShapeDtypeStruct((Q, C), jnp.float32),
            jax.ShapeDtypeStruct((1, NL), jnp.float32),
        ],
        scratch_shapes=[
            pltpu.VMEM((Q, 16), jnp.float32),
            pltpu.VMEM((Q, 16), jnp.float32),
        ],
        compiler_params=cparams,
    )(x, f, labs, t)

    def _fallback(_):
        return pl.pallas_call(
            functools.partial(_body_full, n, g),
            grid=(g,),
            in_specs=[
                pl.BlockSpec((Q, D), lambda i: (0, 0)),
                pl.BlockSpec((BN, D), lambda i: (i, 0)),
                pl.BlockSpec((1, 1, BN), lambda i: (i, 0, 0)),
            ],
            out_specs=pl.BlockSpec((Q, C), lambda i: (0, 0)),
            out_shape=jax.ShapeDtypeStruct((Q, C), jnp.float32),
            scratch_shapes=[
                pltpu.VMEM((Q, 16), jnp.float32),
                pltpu.VMEM((Q, 16), jnp.float32),
            ],
            compiler_params=cparams,
        )(x, f, labs)

    return lax.cond(jnp.max(flag) >= 2.5, _fallback, lambda _: out, 0)


# speedup vs baseline: 1.7563x; 1.6467x over previous
import functools

import jax
import jax.numpy as jnp
from jax import lax
from jax.experimental import pallas as pl
from jax.experimental.pallas import tpu as pltpu

Q = 1024
D = 32
K = 5
C = 10
BN = 2048
S = 16
NL = BN // S
INF = float("inf")


def _dist2(x_ref, f_ref, g, n_total):
    xx = x_ref[:]
    xn = jnp.sum(xx * xx, axis=1, keepdims=True)
    f = f_ref[:]
    yn = jnp.sum(f * f, axis=1).reshape(1, BN)
    col = lax.broadcasted_iota(jnp.int32, (1, BN), 1)
    gcol = g * BN + col
    yn = jnp.where(gcol < n_total, yn, INF)
    prod = lax.dot_general(xx, f, (((1,), (1,)), ((), ())),
                           preferred_element_type=jnp.float32)
    d2 = jnp.maximum(xn + yn - 2.0 * prod, 0.0)
    return d2, gcol


def _finish_out(n_d, n_p, out_ref):
    dist = [jnp.sqrt(v) for v in n_d]
    s0 = -dist[0]
    e = [jnp.exp(-v - s0) for v in dist]
    tot = e[0] + e[1] + e[2] + e[3] + e[4]
    iota_c = lax.broadcasted_iota(jnp.int32, (Q, C), 1)
    o = jnp.zeros((Q, C), jnp.float32)
    for j in range(K):
        labj = n_p[j].astype(jnp.int32) & 15
        o = o + (e[j] / tot) * (labj == iota_c).astype(jnp.float32)
    out_ref[:] = o


def _merge_run(bw_d, bw_p, run_d, run_p):
    cd = jnp.concatenate([run_d[:, :8], bw_d[:, :8]], axis=1)
    cp = jnp.concatenate([run_p[:, :8], bw_p[:, :8]], axis=1)
    n_d, n_p = [], []
    for j in range(K):
        m = jnp.min(cd, axis=1, keepdims=True)
        sel = jnp.min(jnp.where(cd == m, cp, INF), axis=1, keepdims=True)
        n_d.append(m)
        n_p.append(sel)
        if j < K - 1:
            cd = jnp.where(cp == sel, INF, cd)
    inf1 = jnp.full((Q, 1), INF, jnp.float32)
    run_d[:] = jnp.concatenate(n_d + [inf1] * 11, axis=1)
    run_p[:] = jnp.concatenate(n_p + [inf1] * 11, axis=1)
    return n_d, n_p


def _extract5(dv, pv, width_pad):
    out_d, out_p = [], []
    for j in range(K):
        m = jnp.min(dv, axis=1, keepdims=True)
        sel = jnp.min(jnp.where(dv == m, pv, INF), axis=1, keepdims=True)
        out_d.append(m)
        out_p.append(sel)
        if j < K - 1:
            dv = jnp.where(pv == sel, INF, dv)
    inf1 = jnp.full((Q, 1), INF, jnp.float32)
    return (jnp.concatenate(out_d + [inf1] * width_pad, axis=1),
            jnp.concatenate(out_p + [inf1] * width_pad, axis=1))


def _body_a(n_total, n_blocks, x_ref, f_ref, out_ref, runv):
    g = pl.program_id(0)

    @pl.when(g == 0)
    def _init():
        runv[:] = jnp.full((Q, 16), INF, jnp.float32)

    d2, _ = _dist2(x_ref, f_ref, g, n_total)
    cm = d2[:, :NL]
    for j in range(1, S):
        cm = jnp.minimum(cm, d2[:, j * NL:(j + 1) * NL])
    c = jnp.concatenate([runv[:, :8], cm], axis=1)
    ms = []
    for j in range(K):
        m = jnp.min(c, axis=1, keepdims=True)
        ms.append(m)
        if j < K - 1:
            c = jnp.where(c <= m, INF, c)
    inf1 = jnp.full((Q, 1), INF, jnp.float32)
    runv[:] = jnp.concatenate(ms + [inf1] * 11, axis=1)

    @pl.when(g == n_blocks - 1)
    def _finish():
        out_ref[:] = ms[K - 1]


def _body_b(n_total, n_blocks, x_ref, f_ref, lab_ref, t_ref, out_ref,
            flag_ref, run_d, run_p):
    g = pl.program_id(0)

    @pl.when(g == 0)
    def _init():
        run_d[:] = jnp.full((Q, 16), INF, jnp.float32)
        run_p[:] = jnp.full((Q, 16), INF, jnp.float32)
        flag_ref[:] = jnp.zeros((1, NL), jnp.float32)

    d2, gcol = _dist2(x_ref, f_ref, g, n_total)
    lab = lab_ref[0]
    pk = (gcol * 16 + lab).astype(jnp.float32)

    t = t_ref[:] * 1.00002

    hms = [jnp.where(d2[:, j * NL:(j + 1) * NL] <= t,
                     d2[:, j * NL:(j + 1) * NL], INF) for j in range(S)]
    pks = [pk[:, j * NL:(j + 1) * NL] for j in range(S)]
    cnt = jnp.zeros((Q, NL), jnp.float32)
    cm = jnp.full((Q, NL), INF, jnp.float32)
    for j in range(S):
        cnt = cnt + (hms[j] < INF).astype(jnp.float32)
        cm = jnp.minimum(cm, hms[j])
    flag_ref[:] = jnp.maximum(flag_ref[:],
                              jnp.max(cnt, axis=0, keepdims=True))
    cp = jnp.full((Q, NL), INF, jnp.float32)
    for j in range(S):
        cp = jnp.minimum(cp, jnp.where(hms[j] == cm, pks[j], INF))
    cm2 = jnp.full((Q, NL), INF, jnp.float32)
    hms2 = [jnp.where(pks[j] == cp, INF, hms[j]) for j in range(S)]
    for j in range(S):
        cm2 = jnp.minimum(cm2, hms2[j])
    cp2 = jnp.full((Q, NL), INF, jnp.float32)
    for j in range(S):
        cp2 = jnp.minimum(cp2, jnp.where(hms2[j] == cm2, pks[j], INF))

    bw_d, bw_p = _extract5(jnp.concatenate([cm, cm2], axis=1),
                           jnp.concatenate([cp, cp2], axis=1), 3)
    n_d, n_p = _merge_run(bw_d, bw_p, run_d, run_p)

    @pl.when(g == n_blocks - 1)
    def _finish():
        _finish_out(n_d, n_p, out_ref)


def _body_full(n_total, n_blocks, x_ref, f_ref, lab_ref, out_ref,
               run_d, run_p):
    g = pl.program_id(0)

    @pl.when(g == 0)
    def _init():
        run_d[:] = jnp.full((Q, 16), INF, jnp.float32)
        run_p[:] = jnp.full((Q, 16), INF, jnp.float32)

    d2, gcol = _dist2(x_ref, f_ref, g, n_total)
    lab = lab_ref[0]
    pk = (gcol * 16 + lab).astype(jnp.float32)
    bw_d, bw_p = _extract5(d2, jnp.broadcast_to(pk, (Q, BN)), 3)
    n_d, n_p = _merge_run(bw_d, bw_p, run_d, run_p)

    @pl.when(g == n_blocks - 1)
    def _finish():
        _finish_out(n_d, n_p, out_ref)


def kernel(x, train_features, train_labels):
    n = train_features.shape[0]
    g = -(-n // BN)
    npad = g * BN
    f = jnp.pad(train_features, ((0, npad - n), (0, 0)))
    labs = jnp.pad(train_labels, (0, npad - n)).reshape(g, 1, BN)
    cparams = pltpu.CompilerParams(dimension_semantics=("arbitrary",))

    t = pl.pallas_call(
        functools.partial(_body_a, n, g),
        grid=(g,),
        in_specs=[
            pl.BlockSpec((Q, D), lambda i: (0, 0)),
            pl.BlockSpec((BN, D), lambda i: (i, 0)),
        ],
        out_specs=pl.BlockSpec((Q, 1), lambda i: (0, 0)),
        out_shape=jax.ShapeDtypeStruct((Q, 1), jnp.float32),
        scratch_shapes=[pltpu.VMEM((Q, 16), jnp.float32)],
        compiler_params=cparams,
    )(x, f)

    out, flag = pl.pallas_call(
        functools.partial(_body_b, n, g),
        grid=(g,),
        in_specs=[
            pl.BlockSpec((Q, D), lambda i: (0, 0)),
            pl.BlockSpec((BN, D), lambda i: (i, 0)),
            pl.BlockSpec((1, 1, BN), lambda i: (i, 0, 0)),
            pl.BlockSpec((Q, 1), lambda i: (0, 0)),
        ],
        out_specs=[
            pl.BlockSpec((Q, C), lambda i: (0, 0)),
            pl.BlockSpec((1, NL), lambda i: (0, 0)),
        ],
        out_shape=[
            jax.ShapeDtypeStruct((Q, C), jnp.float32),
            jax.ShapeDtypeStruct((1, NL), jnp.float32),
        ],
        scratch_shapes=[
            pltpu.VMEM((Q, 16), jnp.float32),
            pltpu.VMEM((Q, 16), jnp.float32),
        ],
        compiler_params=cparams,
    )(x, f, labs, t)

    def _fallback(_):
        return pl.pallas_call(
            functools.partial(_body_full, n, g),
            grid=(g,),
            in_specs=[
                pl.BlockSpec((Q, D), lambda i: (0, 0)),
                pl.BlockSpec((BN, D), lambda i: (i, 0)),
                pl.BlockSpec((1, 1, BN), lambda i: (i, 0, 0)),
            ],
            out_specs=pl.BlockSpec((Q, C), lambda i: (0, 0)),
            out_shape=jax.ShapeDtypeStruct((Q, C), jnp.float32),
            scratch_shapes=[
                pltpu.VMEM((Q, 16), jnp.float32),
                pltpu.VMEM((Q, 16), jnp.float32),
            ],
            compiler_params=cparams,
        )(x, f, labs)

    return lax.cond(jnp.max(flag) >= 2.5, _fallback, lambda _: out, 0)


# speedup vs baseline: 2.0402x; 1.1616x over previous
import functools

import jax
import jax.numpy as jnp
from jax import lax
from jax.experimental import pallas as pl
from jax.experimental.pallas import tpu as pltpu
from jax.experimental.pallas import tpu_sc as plsc

Q = 1024
D = 32
K = 5
C = 10
BN = 2048
INF = float("inf")

_NW = 32
_QW = Q // _NW
_NG = _QW // 16


def _body(n_total, n_blocks, x_ref, f_ref, lab_ref, d_out, l_out,
          run_d, run_p):
    g = pl.program_id(0)

    @pl.when(g == 0)
    def _init():
        run_d[:] = jnp.full((Q, 16), INF, jnp.float32)
        run_p[:] = jnp.full((Q, 16), INF, jnp.float32)

    xx = x_ref[:]
    xn = jnp.sum(xx * xx, axis=1, keepdims=True)
    f = f_ref[:]
    yn = jnp.sum(f * f, axis=1).reshape(1, BN)
    col = lax.broadcasted_iota(jnp.int32, (1, BN), 1)
    gcol = g * BN + col
    yn = jnp.where(gcol < n_total, yn, INF)
    prod = lax.dot_general(xx, f, (((1,), (1,)), ((), ())),
                           preferred_element_type=jnp.float32)
    d2 = jnp.maximum(xn + yn - 2.0 * prod, 0.0)

    lab = lab_ref[0]
    pk = (gcol * 16 + lab).astype(jnp.float32)

    bw_d, bw_p = [], []
    d = d2
    for j in range(K):
        m = jnp.min(d, axis=1, keepdims=True)
        sel = jnp.min(jnp.where(d == m, pk, INF),
                      axis=1, keepdims=True)
        bw_d.append(m)
        bw_p.append(sel)
        if j < K - 1:
            d = jnp.where(pk == sel, INF, d)

    inf1 = jnp.full((Q, 1), INF, jnp.float32)
    cd = jnp.concatenate([run_d[:, :8]] + bw_d + [inf1] * 3, axis=1)
    cp = jnp.concatenate([run_p[:, :8]] + bw_p + [inf1] * 3, axis=1)
    n_d, n_p = [], []
    for j in range(K):
        m = jnp.min(cd, axis=1, keepdims=True)
        sel = jnp.min(jnp.where(cd == m, cp, INF), axis=1, keepdims=True)
        n_d.append(m)
        n_p.append(sel)
        if j < K - 1:
            cd = jnp.where(cp == sel, INF, cd)
    run_d[:] = jnp.concatenate(n_d + [inf1] * 11, axis=1)
    run_p[:] = jnp.concatenate(n_p + [inf1] * 11, axis=1)

    @pl.when(g == n_blocks - 1)
    def _finish():
        dist = [jnp.sqrt(v) for v in n_d]
        labf = [(p.astype(jnp.int32) & 15).astype(jnp.float32) for p in n_p]
        zero1 = jnp.zeros((Q, 1), jnp.float32)
        d_out[:] = jnp.concatenate(dist + [zero1] * 11, axis=1)
        l_out[:] = jnp.concatenate(labf + [zero1] * 11, axis=1)


def _sc_combine(d_hbm, l_hbm, out_hbm, d_v, l_v, o_v):
    wid = lax.axis_index("s") * 2 + lax.axis_index("c")
    qbase = wid * _QW
    for j in range(K):
        pltpu.sync_copy(d_hbm.at[pl.ds(j * Q + qbase, _QW)],
                        d_v.at[pl.ds(j * _QW, _QW)])
        pltpu.sync_copy(l_hbm.at[pl.ds(j * Q + qbase, _QW)],
                        l_v.at[pl.ds(j * _QW, _QW)])
    def per_group(grp, carry):
        o16 = grp * 16
        ds = [d_v[pl.ds(j * _QW + o16, 16)] for j in range(K)]
        mn = ds[0]
        es = [jnp.exp(mn - dj) for dj in ds]
        tot = es[0] + es[1] + es[2] + es[3] + es[4]
        ws = [e / tot for e in es]
        ls = [l_v[pl.ds(j * _QW + o16, 16)] for j in range(K)]
        for c in range(C):
            oc = jnp.zeros((16,), jnp.float32)
            for j in range(K):
                oc = oc + jnp.where(ls[j] == float(c), ws[j], 0.0)
            o_v[pl.ds(c * _QW + o16, 16)] = oc
        return carry

    lax.fori_loop(0, _NG, per_group, 0)
    for c in range(C):
        pltpu.sync_copy(o_v.at[pl.ds(c * _QW, _QW)],
                        out_hbm.at[pl.ds(c * Q + qbase, _QW)])


def kernel(x, train_features, train_labels):
    n = train_features.shape[0]
    g = -(-n // BN)
    npad = g * BN
    f = jnp.pad(train_features, ((0, npad - n), (0, 0)))
    labs = jnp.pad(train_labels, (0, npad - n)).reshape(g, 1, BN)

    cand_d, cand_l = pl.pallas_call(
        functools.partial(_body, n, g),
        grid=(g,),
        in_specs=[
            pl.BlockSpec((Q, D), lambda i: (0, 0)),
            pl.BlockSpec((BN, D), lambda i: (i, 0)),
            pl.BlockSpec((1, 1, BN), lambda i: (i, 0, 0)),
        ],
        out_specs=[
            pl.BlockSpec((Q, 16), lambda i: (0, 0)),
            pl.BlockSpec((Q, 16), lambda i: (0, 0)),
        ],
        out_shape=[
            jax.ShapeDtypeStruct((Q, 16), jnp.float32),
            jax.ShapeDtypeStruct((Q, 16), jnp.float32),
        ],
        scratch_shapes=[
            pltpu.VMEM((Q, 16), jnp.float32),
            pltpu.VMEM((Q, 16), jnp.float32),
        ],
        compiler_params=pltpu.CompilerParams(
            dimension_semantics=("arbitrary",),
        ),
    )(x, f, labs)

    d_t = cand_d[:, :K].T.reshape(K * Q)
    l_t = cand_l[:, :K].T.reshape(K * Q)

    mesh = plsc.VectorSubcoreMesh(core_axis_name="c", subcore_axis_name="s")
    outcm = functools.partial(
        pl.kernel, mesh=mesh,
        out_type=jax.ShapeDtypeStruct((C * Q,), jnp.float32),
        scratch_types=[
            pltpu.VMEM((K * _QW,), jnp.float32),
            pltpu.VMEM((K * _QW,), jnp.float32),
            pltpu.VMEM((C * _QW,), jnp.float32),
        ],
    )(_sc_combine)(d_t, l_t)

    return outcm.reshape(C, Q).T
